# baseline (device time: 23803 ns/iter reference)
import jax
import jax.numpy as jnp
from jax import lax
from jax.experimental import pallas as pl
from jax.experimental.pallas import tpu as pltpu


def kernel(x, dy):
    k_per, d = x.shape
    _, f = dy.shape
    out_rows = d // 2
    half = out_rows // 2

    def body(x_ref, dy_ref, out_ref, partial_ref, send_ref, recv1_ref,
             recv2_ref, sem_send1, sem_recv1, sem_send2, sem_recv2):
        my_x = lax.axis_index("x")
        my_y = lax.axis_index("y")
        x_nbr = (1 - my_x, my_y)
        y_nbr = (my_x, 1 - my_y)

        barrier_sem = pltpu.get_barrier_semaphore()
        for nbr in (x_nbr, y_nbr):
            pl.semaphore_signal(
                barrier_sem, inc=1,
                device_id=nbr, device_id_type=pl.DeviceIdType.MESH,
            )
        pl.semaphore_wait(barrier_sem, 2)

        xb = x_ref[...].astype(jnp.bfloat16)
        dyb = dy_ref[...].astype(jnp.bfloat16)
        partial_ref[...] = lax.dot_general(
            xb, dyb, (((0,), (0,)), ((), ())),
            preferred_element_type=jnp.float32,
        )

        send_off = (1 - my_x) * out_rows + my_y * half
        send_ref[...] = partial_ref[pl.ds(send_off, half), :].astype(jnp.bfloat16)
        rdma1 = pltpu.make_async_remote_copy(
            src_ref=send_ref, dst_ref=recv1_ref,
            send_sem=sem_send1, recv_sem=sem_recv1,
            device_id=x_nbr, device_id_type=pl.DeviceIdType.MESH,
        )
        rdma1.start()
        rdma1.wait()

        rdma2 = pltpu.make_async_remote_copy(
            src_ref=recv1_ref, dst_ref=recv2_ref,
            send_sem=sem_send2, recv_sem=sem_recv2,
            device_id=y_nbr, device_id_type=pl.DeviceIdType.MESH,
        )
        rdma2.start()
        rdma2.wait()

        base = my_x * out_rows
        off1 = my_y * half
        off2 = (1 - my_y) * half
        out_ref[pl.ds(off1, half), :] = (
            partial_ref[pl.ds(base + off1, half), :]
            + recv1_ref[...].astype(jnp.float32)
        )
        out_ref[pl.ds(off2, half), :] = (
            partial_ref[pl.ds(base + off2, half), :]
            + recv2_ref[...].astype(jnp.float32)
        )

    return pl.pallas_call(
        body,
        out_shape=jax.ShapeDtypeStruct((out_rows, f), jnp.float32),
        in_specs=[
            pl.BlockSpec(memory_space=pltpu.VMEM),
            pl.BlockSpec(memory_space=pltpu.VMEM),
        ],
        out_specs=pl.BlockSpec(memory_space=pltpu.VMEM),
        scratch_shapes=[
            pltpu.VMEM((d, f), jnp.float32),
            pltpu.VMEM((half, f), jnp.bfloat16),
            pltpu.VMEM((half, f), jnp.bfloat16),
            pltpu.VMEM((half, f), jnp.bfloat16),
            pltpu.SemaphoreType.DMA,
            pltpu.SemaphoreType.DMA,
            pltpu.SemaphoreType.DMA,
            pltpu.SemaphoreType.DMA,
        ],
        compiler_params=pltpu.CompilerParams(collective_id=0),
    )(x, dy)


# device time: 18650 ns/iter; 1.2763x vs baseline; 1.2763x over previous
import jax
import jax.numpy as jnp
from jax import lax
from jax.experimental import pallas as pl
from jax.experimental.pallas import tpu as pltpu

S = 4


def kernel(x, dy):
    k_per, d = x.shape
    _, f = dy.shape
    out_rows = d // 2
    half = out_rows // 2
    sub = half // S

    contract = (((0,), (0,)), ((), ()))

    def body(x_ref, dy_ref, out_ref, send_ref, recv1_ref, recv2_ref,
             sems1_send, sems1_recv, sems2_send, sems2_recv):
        my_x = lax.axis_index("x")
        my_y = lax.axis_index("y")
        x_nbr = (1 - my_x, my_y)
        y_nbr = (my_x, 1 - my_y)

        barrier_sem = pltpu.get_barrier_semaphore()
        for nbr in (x_nbr, y_nbr):
            pl.semaphore_signal(
                barrier_sem, inc=1,
                device_id=nbr, device_id_type=pl.DeviceIdType.MESH,
            )
        pl.semaphore_wait(barrier_sem, 2)

        dyb = dy_ref[...].astype(jnp.bfloat16)

        send_off = (1 - my_x) * out_rows + my_y * half
        xs = x_ref[:, pl.ds(send_off, half)].astype(jnp.bfloat16)
        send_ref[...] = lax.dot_general(
            xs, dyb, contract, preferred_element_type=jnp.float32,
        ).astype(jnp.bfloat16)
        rdma1 = []
        for s in range(S):
            r = pltpu.make_async_remote_copy(
                src_ref=send_ref.at[pl.ds(s * sub, sub)],
                dst_ref=recv1_ref.at[pl.ds(s * sub, sub)],
                send_sem=sems1_send.at[s], recv_sem=sems1_recv.at[s],
                device_id=x_nbr, device_id_type=pl.DeviceIdType.MESH,
            )
            r.start()
            rdma1.append(r)

        base = my_x * out_rows
        xo = x_ref[:, pl.ds(base, out_rows)].astype(jnp.bfloat16)
        out_ref[...] = lax.dot_general(
            xo, dyb, contract, preferred_element_type=jnp.float32,
        )

        off1 = my_y * half
        off2 = (1 - my_y) * half
        rdma2 = []
        for s in range(S):
            rdma1[s].wait_recv()
            r = pltpu.make_async_remote_copy(
                src_ref=recv1_ref.at[pl.ds(s * sub, sub)],
                dst_ref=recv2_ref.at[pl.ds(s * sub, sub)],
                send_sem=sems2_send.at[s], recv_sem=sems2_recv.at[s],
                device_id=y_nbr, device_id_type=pl.DeviceIdType.MESH,
            )
            r.start()
            rdma2.append(r)
            rows = pl.ds(off1 + s * sub, sub)
            out_ref[rows, :] = (
                out_ref[rows, :]
                + recv1_ref[pl.ds(s * sub, sub), :].astype(jnp.float32)
            )

        for s in range(S):
            rdma2[s].wait_recv()
            rows = pl.ds(off2 + s * sub, sub)
            out_ref[rows, :] = (
                out_ref[rows, :]
                + recv2_ref[pl.ds(s * sub, sub), :].astype(jnp.float32)
            )

        for s in range(S):
            rdma1[s].wait_send()
            rdma2[s].wait_send()

    return pl.pallas_call(
        body,
        out_shape=jax.ShapeDtypeStruct((out_rows, f), jnp.float32),
        in_specs=[
            pl.BlockSpec(memory_space=pltpu.VMEM),
            pl.BlockSpec(memory_space=pltpu.VMEM),
        ],
        out_specs=pl.BlockSpec(memory_space=pltpu.VMEM),
        scratch_shapes=[
            pltpu.VMEM((half, f), jnp.bfloat16),
            pltpu.VMEM((half, f), jnp.bfloat16),
            pltpu.VMEM((half, f), jnp.bfloat16),
            pltpu.SemaphoreType.DMA((S,)),
            pltpu.SemaphoreType.DMA((S,)),
            pltpu.SemaphoreType.DMA((S,)),
            pltpu.SemaphoreType.DMA((S,)),
        ],
        compiler_params=pltpu.CompilerParams(collective_id=0),
    )(x, dy)


# device time: 17746 ns/iter; 1.3413x vs baseline; 1.0509x over previous
import jax
import jax.numpy as jnp
from jax import lax
from jax.experimental import pallas as pl
from jax.experimental.pallas import tpu as pltpu

S = 8


def kernel(x, dy):
    k_per, d = x.shape
    _, f = dy.shape
    out_rows = d // 2
    half = out_rows // 2
    sub = half // S

    contract = (((0,), (0,)), ((), ()))

    def body(x_ref, dy_ref, out_ref, send_ref, recv1_ref, recv2_ref,
             sems1_send, sems1_recv, sems2_send, sems2_recv):
        my_x = lax.axis_index("x")
        my_y = lax.axis_index("y")
        x_nbr = (1 - my_x, my_y)
        y_nbr = (my_x, 1 - my_y)

        barrier_sem = pltpu.get_barrier_semaphore()
        for nbr in (x_nbr, y_nbr):
            pl.semaphore_signal(
                barrier_sem, inc=1,
                device_id=nbr, device_id_type=pl.DeviceIdType.MESH,
            )
        pl.semaphore_wait(barrier_sem, 2)

        dyb = dy_ref[...].astype(jnp.bfloat16)

        send_off = (1 - my_x) * out_rows + my_y * half
        xs = x_ref[:, pl.ds(send_off, half)].astype(jnp.bfloat16)
        send_ref[...] = lax.dot_general(
            xs, dyb, contract, preferred_element_type=jnp.float32,
        ).astype(jnp.bfloat16)
        rdma1 = []
        for s in range(S):
            r = pltpu.make_async_remote_copy(
                src_ref=send_ref.at[pl.ds(s * sub, sub)],
                dst_ref=recv1_ref.at[pl.ds(s * sub, sub)],
                send_sem=sems1_send.at[s], recv_sem=sems1_recv.at[s],
                device_id=x_nbr, device_id_type=pl.DeviceIdType.MESH,
            )
            r.start()
            rdma1.append(r)

        base = my_x * out_rows
        xo = x_ref[:, pl.ds(base, out_rows)].astype(jnp.bfloat16)
        out_ref[...] = lax.dot_general(
            xo, dyb, contract, preferred_element_type=jnp.float32,
        )

        off1 = my_y * half
        off2 = (1 - my_y) * half
        rdma2 = []
        for s in range(S):
            rdma1[s].wait_recv()
            r = pltpu.make_async_remote_copy(
                src_ref=recv1_ref.at[pl.ds(s * sub, sub)],
                dst_ref=recv2_ref.at[pl.ds(s * sub, sub)],
                send_sem=sems2_send.at[s], recv_sem=sems2_recv.at[s],
                device_id=y_nbr, device_id_type=pl.DeviceIdType.MESH,
            )
            r.start()
            rdma2.append(r)
            rows = pl.ds(off1 + s * sub, sub)
            out_ref[rows, :] = (
                out_ref[rows, :]
                + recv1_ref[pl.ds(s * sub, sub), :].astype(jnp.float32)
            )

        for s in range(S):
            rdma2[s].wait_recv()
            rows = pl.ds(off2 + s * sub, sub)
            out_ref[rows, :] = (
                out_ref[rows, :]
                + recv2_ref[pl.ds(s * sub, sub), :].astype(jnp.float32)
            )

        for s in range(S):
            rdma1[s].wait_send()
            rdma2[s].wait_send()

    return pl.pallas_call(
        body,
        out_shape=jax.ShapeDtypeStruct((out_rows, f), jnp.float32),
        in_specs=[
            pl.BlockSpec(memory_space=pltpu.VMEM),
            pl.BlockSpec(memory_space=pltpu.VMEM),
        ],
        out_specs=pl.BlockSpec(memory_space=pltpu.VMEM),
        scratch_shapes=[
            pltpu.VMEM((half, f), jnp.bfloat16),
            pltpu.VMEM((half, f), jnp.bfloat16),
            pltpu.VMEM((half, f), jnp.bfloat16),
            pltpu.SemaphoreType.DMA((S,)),
            pltpu.SemaphoreType.DMA((S,)),
            pltpu.SemaphoreType.DMA((S,)),
            pltpu.SemaphoreType.DMA((S,)),
        ],
        compiler_params=pltpu.CompilerParams(collective_id=0),
    )(x, dy)
